# baseline (speedup 1.0000x reference)
import jax
import jax.numpy as jnp
from jax.experimental import pallas as pl
from jax.experimental.pallas import tpu as pltpu

L = 64
N = 32
NHID = 1024
Q = 64
B = 16
TOPK = 8
BO = 8
NOCT = B // BO
NH = 4
NHALF = N // NH
LCH = 64
SCALE = 1.0 / 32.0


def _attn_kernel(k_ref, qt_ref, att_ref, idx_ref, smax_ref):
    nh = pl.program_id(1)
    lc = pl.program_id(2)
    a = k_ref[0, :, 0].reshape(NHALF * BO, LCH * NHID).astype(jnp.bfloat16)
    qt = qt_ref[...]
    parts = [
        jax.lax.dot_general(
            a[:, i * NHID:(i + 1) * NHID], qt, (((1,), (0,)), ((), ())),
            preferred_element_type=jnp.float32,
            precision=jax.lax.Precision.DEFAULT,
        )
        for i in range(LCH)
    ]
    s = parts[0]
    for p in parts[1:]:
        s = jnp.maximum(s, p)
    rows = pl.ds(nh * NHALF * BO, NHALF * BO)

    @pl.when(lc == 0)
    def _():
        smax_ref[rows, :] = s

    @pl.when(lc > 0)
    def _():
        smax_ref[rows, :] = jnp.maximum(smax_ref[rows, :], s)

    @pl.when((lc == pl.num_programs(2) - 1) & (nh == pl.num_programs(1) - 1))
    def _():
        sm3 = smax_ref[...].reshape(N, BO, BO * Q)
        atts, idxs = [], []
        iota = jax.lax.broadcasted_iota(jnp.int32, (N, Q), 0)
        for bo in range(BO):
            logits = sm3[:, bo, bo * Q:(bo + 1) * Q] * SCALE
            m = jnp.max(logits, axis=0, keepdims=True)
            e = jnp.exp(logits - m)
            att = e / jnp.sum(e, axis=0, keepdims=True)
            atts.append(att)
            vals = att
            rows_k = []
            for _ in range(TOPK):
                cur = jnp.max(vals, axis=0, keepdims=True)
                idx = jnp.min(jnp.where(vals >= cur, iota, N), axis=0)
                rows_k.append(idx)
                vals = jnp.where(iota == idx[None, :], -jnp.inf, vals)
            idxs.append(jnp.stack(rows_k, axis=0))
        att_ref[...] = jnp.stack(atts, axis=0)
        idx_ref[...] = jnp.stack(idxs, axis=0)


def kernel(query, keys):
    k5 = keys.reshape(NH, NHALF, NOCT, BO, L * NHID)
    qt = jnp.transpose(query, (1, 2, 0)).reshape(NHID, B * Q)
    qt = qt.astype(jnp.bfloat16)
    att_bnq, idx_bkq = pl.pallas_call(
        _attn_kernel,
        grid=(NOCT, NH, L // LCH),
        in_specs=[
            pl.BlockSpec((1, NHALF, 1, BO, LCH * NHID),
                         lambda o, nh, lc: (nh, 0, o, 0, lc)),
            pl.BlockSpec((NHID, BO * Q), lambda o, nh, lc: (0, o)),
        ],
        out_specs=[
            pl.BlockSpec((BO, N, Q), lambda o, nh, lc: (o, 0, 0)),
            pl.BlockSpec((BO, TOPK, Q), lambda o, nh, lc: (o, 0, 0)),
        ],
        out_shape=[
            jax.ShapeDtypeStruct((B, N, Q), jnp.float32),
            jax.ShapeDtypeStruct((B, TOPK, Q), jnp.int32),
        ],
        scratch_shapes=[pltpu.VMEM((N * BO, BO * Q), jnp.float32)],
    )(k5, qt)
    attention = jnp.transpose(att_bnq, (2, 0, 1))
    topk_indices = jnp.transpose(idx_bkq, (1, 2, 0))
    return (attention, topk_indices)


# speedup vs baseline: 1.8354x; 1.8354x over previous
import jax
import jax.numpy as jnp
from jax.experimental import pallas as pl
from jax.experimental.pallas import tpu as pltpu

L = 64
N = 32
NHID = 1024
Q = 64
B = 16
TOPK = 8
BO = 8
NOCT = B // BO
NH = 2
NHALF = N // NH
LCH = 32
SCALE = 1.0 / 32.0


def _attn_kernel(k_ref, qt_ref, att_ref, idx_ref, smax_ref):
    nh = pl.program_id(1)
    lc = pl.program_id(2)
    a = k_ref[0, :, 0].reshape(NHALF * BO, LCH * NHID).astype(jnp.bfloat16)
    qt = qt_ref[...]
    parts = [
        jax.lax.dot_general(
            a[:, i * NHID:(i + 1) * NHID], qt, (((1,), (0,)), ((), ())),
            preferred_element_type=jnp.float32,
            precision=jax.lax.Precision.DEFAULT,
        )
        for i in range(LCH)
    ]
    s = parts[0]
    for p in parts[1:]:
        s = jnp.maximum(s, p)
    rows = pl.ds(nh * NHALF * BO, NHALF * BO)

    @pl.when(lc == 0)
    def _():
        smax_ref[rows, :] = s

    @pl.when(lc > 0)
    def _():
        smax_ref[rows, :] = jnp.maximum(smax_ref[rows, :], s)

    @pl.when((lc == pl.num_programs(2) - 1) & (nh == pl.num_programs(1) - 1))
    def _():
        sm3 = smax_ref[...].reshape(N, BO, BO * Q)
        atts, idxs = [], []
        iota = jax.lax.broadcasted_iota(jnp.int32, (N, Q), 0)
        for bo in range(BO):
            logits = sm3[:, bo, bo * Q:(bo + 1) * Q] * SCALE
            m = jnp.max(logits, axis=0, keepdims=True)
            e = jnp.exp(logits - m)
            att = e / jnp.sum(e, axis=0, keepdims=True)
            atts.append(att)
            vals = att
            rows_k = []
            for _ in range(TOPK):
                cur = jnp.max(vals, axis=0, keepdims=True)
                idx = jnp.min(jnp.where(vals >= cur, iota, N), axis=0)
                rows_k.append(idx)
                vals = jnp.where(iota == idx[None, :], -jnp.inf, vals)
            idxs.append(jnp.stack(rows_k, axis=0))
        att_ref[...] = jnp.stack(atts, axis=0)
        idx_ref[...] = jnp.stack(idxs, axis=0)


def kernel(query, keys):
    k5 = keys.reshape(NH, NHALF, NOCT, BO, L * NHID)
    qt = jnp.transpose(query, (1, 2, 0)).reshape(NHID, B * Q)
    qt = qt.astype(jnp.bfloat16)
    att_bnq, idx_bkq = pl.pallas_call(
        _attn_kernel,
        grid=(NOCT, NH, L // LCH),
        in_specs=[
            pl.BlockSpec((1, NHALF, 1, BO, LCH * NHID),
                         lambda o, nh, lc: (nh, 0, o, 0, lc)),
            pl.BlockSpec((NHID, BO * Q), lambda o, nh, lc: (0, o)),
        ],
        out_specs=[
            pl.BlockSpec((BO, N, Q), lambda o, nh, lc: (o, 0, 0)),
            pl.BlockSpec((BO, TOPK, Q), lambda o, nh, lc: (o, 0, 0)),
        ],
        out_shape=[
            jax.ShapeDtypeStruct((B, N, Q), jnp.float32),
            jax.ShapeDtypeStruct((B, TOPK, Q), jnp.int32),
        ],
        scratch_shapes=[pltpu.VMEM((N * BO, BO * Q), jnp.float32)],
    )(k5, qt)
    attention = jnp.transpose(att_bnq, (2, 0, 1))
    topk_indices = jnp.transpose(idx_bkq, (1, 2, 0))
    return (attention, topk_indices)
